# Initial kernel scaffold; baseline (speedup 1.0000x reference)
import jax
import jax.numpy as jnp
from jax.experimental import pallas as pl


def kernel(x, w0, b0, w1, b1, gamma1, beta1, w2, b2, gamma2, beta2, w3, b3, gamma3, beta3, w4, b4):
    raise NotImplementedError("write your pallas kernel here")


# speedup vs baseline: 1.9022x; 1.9022x over previous
import functools

import jax
import jax.numpy as jnp
from jax import lax
from jax.experimental import pallas as pl
from jax.experimental.pallas import tpu as pltpu

LANE = 128
SLOPE = 0.2
BN_EPS = 1e-5
KW = 4
PADW = 2
VMEM_LIMIT = 60 * 1024 * 1024


def _ru(x, m):
    return (x + m - 1) // m * m


def _conv_acc(x_ref, w_ref, keff, ho, wot):
    ce = x_ref.shape[-1]
    cp = w_ref.shape[-1]
    if ce <= 32:
        parts = [x_ref[a:a + ho, b:b + wot, :].reshape(ho * wot, ce)
                 for a in range(keff) for b in range(keff)]
        a2 = jnp.concatenate(parts, axis=1)
        w2 = w_ref[...].reshape(keff * keff * ce, cp)
        return jnp.dot(a2, w2, preferred_element_type=jnp.float32)
    acc = jnp.zeros((ho * wot, cp), jnp.float32)
    for t in range(keff * keff):
        a, b = t // keff, t % keff
        xs = x_ref[a:a + ho, b:b + wot, :].reshape(ho * wot, ce)
        acc = acc + jnp.dot(xs, w_ref[t], preferred_element_type=jnp.float32)
    return acc


def _conv_bias_kernel(x_ref, w_ref, b_ref, o_ref, *, keff, ho, wot, act):
    y = _conv_acc(x_ref, w_ref, keff, ho, wot) + b_ref[...]
    if act == "lrelu":
        y = jnp.where(y >= 0.0, y, y * SLOPE)
    o_ref[...] = y.astype(o_ref.dtype)


def _conv_stats_kernel(x_ref, w_ref, o_ref, ps_ref, pq_ref, *,
                       keff, ho, wo, wot):
    y = _conv_acc(x_ref, w_ref, keff, ho, wot)
    cp = y.shape[-1]
    y3 = y.reshape(ho, wot, cp)
    col = lax.broadcasted_iota(jnp.int32, (1, wot, 1), 1)
    y3 = y3 * (col < wo).astype(jnp.float32)
    ps_ref[...] = jnp.sum(y3, axis=(0, 1)).reshape(1, cp)
    pq_ref[...] = jnp.sum(y3 * y3, axis=(0, 1)).reshape(1, cp)
    o_ref[...] = y3.reshape(ho * wot, cp).astype(o_ref.dtype)


def _prep(x, w, stride):
    n, h, wd, cin = x.shape
    cout = w.shape[0]
    ho = (h + 2 * PADW - KW) // stride + 1
    wo = (wd + 2 * PADW - KW) // stride + 1
    keff = 2 if stride == 2 else KW
    cp = _ru(cout, LANE)
    wot = _ru(wo, 8)
    hpe = ho + keff - 1
    wpe = wot + keff - 1

    x = x.astype(jnp.bfloat16)
    if stride == 2:
        hp2, wp2 = 2 * hpe, 2 * wpe
        xp = jnp.pad(x, ((0, 0), (PADW, hp2 - h - PADW),
                         (PADW, wp2 - wd - PADW), (0, 0)))
        xs = xp.reshape(n, hpe, 2, wpe, 2, cin)
        xs = jnp.transpose(xs, (0, 1, 3, 2, 4, 5))
        xs = xs.reshape(n, hpe, wpe, 4 * cin)
        ce = 4 * cin
    else:
        xs = jnp.pad(x, ((0, 0), (PADW, hpe - h - PADW),
                         (PADW, wpe - wd - PADW), (0, 0)))
        ce = cin
    cep = _ru(ce, 8)
    if cep != ce:
        xs = jnp.pad(xs, ((0, 0), (0, 0), (0, 0), (0, cep - ce)))

    wt = jnp.transpose(w, (2, 3, 1, 0))
    if stride == 2:
        wt = wt.reshape(2, 2, 2, 2, cin, cout)
        wt = jnp.transpose(wt, (0, 2, 1, 3, 4, 5))
        wt = wt.reshape(4, 4 * cin, cout)
    else:
        wt = wt.reshape(16, cin, cout)
    wt = jnp.pad(wt, ((0, 0), (0, cep - ce), (0, cp - cout)))
    wt = wt.astype(jnp.bfloat16)
    return xs, wt, dict(n=n, ho=ho, wo=wo, wot=wot, hpe=hpe, wpe=wpe,
                        cep=cep, cp=cp, cout=cout, keff=keff)


def _conv_layer(x, w, *, stride, bias=None, gamma=None, beta=None,
                act="lrelu", out_dtype=jnp.bfloat16):
    xs, wt, g = _prep(x, w, stride)
    n, ho, wo, wot = g["n"], g["ho"], g["wo"], g["wot"]
    hpe, wpe, cep, cp = g["hpe"], g["wpe"], g["cep"], g["cp"]
    cout, keff = g["cout"], g["keff"]

    grid = (n,)
    x_spec = pl.BlockSpec((None, hpe, wpe, cep), lambda ni: (ni, 0, 0, 0))
    w_spec = pl.BlockSpec((keff * keff, cep, cp), lambda ni: (0, 0, 0))
    o_spec = pl.BlockSpec((None, ho * wot, cp), lambda ni: (ni, 0, 0))
    cparams = pltpu.CompilerParams(
        dimension_semantics=("parallel",), vmem_limit_bytes=VMEM_LIMIT)

    if gamma is not None:
        y2d, ps, pq = pl.pallas_call(
            functools.partial(_conv_stats_kernel, keff=keff, ho=ho, wo=wo,
                              wot=wot),
            out_shape=(jax.ShapeDtypeStruct((n, ho * wot, cp), jnp.bfloat16),
                       jax.ShapeDtypeStruct((n, 1, cp), jnp.float32),
                       jax.ShapeDtypeStruct((n, 1, cp), jnp.float32)),
            grid=grid,
            in_specs=[x_spec, w_spec],
            out_specs=(o_spec,
                       pl.BlockSpec((None, 1, cp), lambda ni: (ni, 0, 0)),
                       pl.BlockSpec((None, 1, cp), lambda ni: (ni, 0, 0))),
            compiler_params=cparams,
        )(xs, wt)

        count = float(n * ho * wo)
        s = jnp.sum(ps[:, 0, :cout], axis=0)
        sq = jnp.sum(pq[:, 0, :cout], axis=0)
        mu = s / count
        var = jnp.maximum(sq / count - mu * mu, 0.0)
        scale = gamma.astype(jnp.float32) * lax.rsqrt(var + BN_EPS)
        shift = beta.astype(jnp.float32) - mu * scale

        y = y2d.reshape(n, ho, wot, cp)[:, :, :wo, :cout]
        y = y.astype(jnp.float32) * scale + shift
        y = jnp.where(y >= 0.0, y, y * SLOPE)
        return y.astype(out_dtype)

    b2d = jnp.pad(bias.astype(jnp.float32), (0, cp - cout)).reshape(1, cp)
    y2d = pl.pallas_call(
        functools.partial(_conv_bias_kernel, keff=keff, ho=ho, wot=wot,
                          act=act),
        out_shape=jax.ShapeDtypeStruct((n, ho * wot, cp), out_dtype),
        grid=grid,
        in_specs=[x_spec, w_spec, pl.BlockSpec((1, cp), lambda ni: (0, 0))],
        out_specs=o_spec,
        compiler_params=cparams,
    )(xs, wt, b2d)
    return y2d.reshape(n, ho, wot, cp)[:, :, :wo, :cout]


def kernel(x, w0, b0, w1, b1, gamma1, beta1, w2, b2, gamma2, beta2,
           w3, b3, gamma3, beta3, w4, b4):
    h = _conv_layer(x, w0, stride=2, bias=b0, act="lrelu")
    h = _conv_layer(h, w1, stride=2, gamma=gamma1, beta=beta1)
    h = _conv_layer(h, w2, stride=2, gamma=gamma2, beta=beta2)
    h = _conv_layer(h, w3, stride=1, gamma=gamma3, beta=beta3)
    h = _conv_layer(h, w4, stride=1, bias=b4, act="none",
                    out_dtype=jnp.float32)
    return h


# speedup vs baseline: 1.9673x; 1.0343x over previous
import functools

import jax
import jax.numpy as jnp
from jax import lax
from jax.experimental import pallas as pl
from jax.experimental.pallas import tpu as pltpu

LANE = 128
SLOPE = 0.2
BN_EPS = 1e-5
KW = 4
PADW = 2
VMEM_LIMIT = 60 * 1024 * 1024


def _ru(x, m):
    return (x + m - 1) // m * m


def _conv_acc(x_ref, w_ref, keff, ho, wot):
    ce = x_ref.shape[-1]
    cp = w_ref.shape[-1]
    if ce <= 32:
        parts = [x_ref[a:a + ho, b:b + wot, :].reshape(ho * wot, ce)
                 for a in range(keff) for b in range(keff)]
        a2 = jnp.concatenate(parts, axis=1)
        w2 = w_ref[...].reshape(keff * keff * ce, cp)
        return jnp.dot(a2, w2, preferred_element_type=jnp.float32)
    acc = jnp.zeros((ho * wot, cp), jnp.float32)
    for t in range(keff * keff):
        a, b = t // keff, t % keff
        xs = x_ref[a:a + ho, b:b + wot, :].reshape(ho * wot, ce)
        acc = acc + jnp.dot(xs, w_ref[t], preferred_element_type=jnp.float32)
    return acc


def _conv_bias_kernel(x_ref, w_ref, b_ref, o_ref, *, keff, ho, wot, act):
    y = _conv_acc(x_ref, w_ref, keff, ho, wot) + b_ref[...]
    if act == "lrelu":
        y = jnp.where(y >= 0.0, y, y * SLOPE)
    o_ref[...] = y.astype(o_ref.dtype)


def _conv_stats_kernel(x_ref, w_ref, o_ref, ps_ref, pq_ref, *,
                       keff, ho, wo, wot):
    y = _conv_acc(x_ref, w_ref, keff, ho, wot)
    cp = y.shape[-1]
    y3 = y.reshape(ho, wot, cp)
    col = lax.broadcasted_iota(jnp.int32, (1, wot, 1), 1)
    y3 = y3 * (col < wo).astype(jnp.float32)
    ps_ref[...] = jnp.sum(y3, axis=(0, 1)).reshape(1, cp)
    pq_ref[...] = jnp.sum(y3 * y3, axis=(0, 1)).reshape(1, cp)
    o_ref[...] = y3.reshape(ho * wot, cp).astype(o_ref.dtype)


def _prep(x, w, stride):
    n, h, wd, cin = x.shape
    cout = w.shape[0]
    ho = (h + 2 * PADW - KW) // stride + 1
    wo = (wd + 2 * PADW - KW) // stride + 1
    keff = 2 if stride == 2 else KW
    cp = _ru(cout, LANE)
    wot = _ru(wo, 8)
    hpe = ho + keff - 1
    wpe = wot + keff - 1

    x = x.astype(jnp.bfloat16)
    if stride == 2:
        hp2, wp2 = 2 * hpe, 2 * wpe
        xp = jnp.pad(x, ((0, 0), (PADW, hp2 - h - PADW),
                         (PADW, wp2 - wd - PADW), (0, 0)))
        xs = xp.reshape(n, hpe, 2, wpe, 2, cin)
        xs = jnp.transpose(xs, (0, 1, 3, 2, 4, 5))
        xs = xs.reshape(n, hpe, wpe, 4 * cin)
        ce = 4 * cin
    else:
        xs = jnp.pad(x, ((0, 0), (PADW, hpe - h - PADW),
                         (PADW, wpe - wd - PADW), (0, 0)))
        ce = cin
    cep = _ru(ce, 8)
    if cep != ce:
        xs = jnp.pad(xs, ((0, 0), (0, 0), (0, 0), (0, cep - ce)))

    wt = jnp.transpose(w, (2, 3, 1, 0))
    if stride == 2:
        wt = wt.reshape(2, 2, 2, 2, cin, cout)
        wt = jnp.transpose(wt, (0, 2, 1, 3, 4, 5))
        wt = wt.reshape(4, 4 * cin, cout)
    else:
        wt = wt.reshape(16, cin, cout)
    wt = jnp.pad(wt, ((0, 0), (0, cep - ce), (0, cp - cout)))
    wt = wt.astype(jnp.bfloat16)
    return xs, wt, dict(n=n, ho=ho, wo=wo, wot=wot, hpe=hpe, wpe=wpe,
                        cep=cep, cp=cp, cout=cout, keff=keff)


def _conv_layer(x, w, *, stride, bias=None, gamma=None, beta=None,
                act="lrelu", out_dtype=jnp.bfloat16):
    xs, wt, g = _prep(x, w, stride)
    n, ho, wo, wot = g["n"], g["ho"], g["wo"], g["wot"]
    hpe, wpe, cep, cp = g["hpe"], g["wpe"], g["cep"], g["cp"]
    cout, keff = g["cout"], g["keff"]

    grid = (n,)
    x_spec = pl.BlockSpec((None, hpe, wpe, cep), lambda ni: (ni, 0, 0, 0))
    w_spec = pl.BlockSpec((keff * keff, cep, cp), lambda ni: (0, 0, 0))
    o_spec = pl.BlockSpec((None, ho * wot, cp), lambda ni: (ni, 0, 0))
    cparams = pltpu.CompilerParams(
        dimension_semantics=("parallel",), vmem_limit_bytes=VMEM_LIMIT)

    if gamma is not None:
        y2d, ps, pq = pl.pallas_call(
            functools.partial(_conv_stats_kernel, keff=keff, ho=ho, wo=wo,
                              wot=wot),
            out_shape=(jax.ShapeDtypeStruct((n, ho * wot, cp), jnp.bfloat16),
                       jax.ShapeDtypeStruct((n, 1, cp), jnp.float32),
                       jax.ShapeDtypeStruct((n, 1, cp), jnp.float32)),
            grid=grid,
            in_specs=[x_spec, w_spec],
            out_specs=(o_spec,
                       pl.BlockSpec((None, 1, cp), lambda ni: (ni, 0, 0)),
                       pl.BlockSpec((None, 1, cp), lambda ni: (ni, 0, 0))),
            compiler_params=cparams,
        )(xs, wt)

        count = float(n * ho * wo)
        s = jnp.sum(ps[:, 0, :cout], axis=0)
        sq = jnp.sum(pq[:, 0, :cout], axis=0)
        mu = s / count
        var = jnp.maximum(sq / count - mu * mu, 0.0)
        scale = gamma.astype(jnp.float32) * lax.rsqrt(var + BN_EPS)
        shift = beta.astype(jnp.float32) - mu * scale

        y = y2d.reshape(n, ho, wot, cp)[:, :, :wo, :cout]
        y = y.astype(jnp.float32) * scale + shift
        y = jnp.where(y >= 0.0, y, y * SLOPE)
        return y.astype(out_dtype)

    b2d = jnp.pad(bias.astype(jnp.float32), (0, cp - cout)).reshape(1, cp)
    y2d = pl.pallas_call(
        functools.partial(_conv_bias_kernel, keff=keff, ho=ho, wot=wot,
                          act=act),
        out_shape=jax.ShapeDtypeStruct((n, ho * wot, cp), out_dtype),
        grid=grid,
        in_specs=[x_spec, w_spec, pl.BlockSpec((1, cp), lambda ni: (0, 0))],
        out_specs=o_spec,
        compiler_params=cparams,
    )(xs, wt, b2d)
    return y2d.reshape(n, ho, wot, cp)[:, :, :wo, :cout]


def _final_kernel(x_ref, w_ref, b_ref, o_ref, *, hpe, wpe, ho, wot):
    ce = x_ref.shape[-1]
    a2 = x_ref[...].reshape(hpe * wpe, ce)
    z = jnp.dot(a2, w_ref[...], preferred_element_type=jnp.float32)
    z3 = z.reshape(hpe, wpe, z.shape[-1])
    y = jnp.zeros((ho, wot), jnp.float32) + b_ref[0, 0]
    for i in range(KW):
        for j in range(KW):
            y = y + z3[i:i + ho, j:j + wot, i * KW + j]
    o_ref[...] = y


def _final_layer(x, w, b):
    n, h, wd, cin = x.shape
    ho = h + 2 * PADW - KW + 1
    wot = _ru(ho, 8)
    hpe = ho + KW - 1
    wpe = _ru(wot + KW - 1, 8)
    xs = jnp.pad(x.astype(jnp.bfloat16),
                 ((0, 0), (PADW, hpe - h - PADW),
                  (PADW, wpe - wd - PADW), (0, 0)))
    wt = jnp.pad(w[0].reshape(cin, KW * KW), ((0, 0), (0, LANE - KW * KW)))
    wt = wt.astype(jnp.bfloat16)
    b2 = b.astype(jnp.float32).reshape(1, 1)

    y = pl.pallas_call(
        functools.partial(_final_kernel, hpe=hpe, wpe=wpe, ho=ho, wot=wot),
        out_shape=jax.ShapeDtypeStruct((n, ho, wot), jnp.float32),
        grid=(n,),
        in_specs=[pl.BlockSpec((None, hpe, wpe, cin), lambda ni: (ni, 0, 0, 0)),
                  pl.BlockSpec((cin, LANE), lambda ni: (0, 0)),
                  pl.BlockSpec((1, 1), lambda ni: (0, 0))],
        out_specs=pl.BlockSpec((None, ho, wot), lambda ni: (ni, 0, 0)),
        compiler_params=pltpu.CompilerParams(
            dimension_semantics=("parallel",), vmem_limit_bytes=VMEM_LIMIT),
    )(xs, wt, b2)
    return y[:, :, :ho].reshape(n, ho, ho, 1)


def kernel(x, w0, b0, w1, b1, gamma1, beta1, w2, b2, gamma2, beta2,
           w3, b3, gamma3, beta3, w4, b4):
    h = _conv_layer(x, w0, stride=2, bias=b0, act="lrelu")
    h = _conv_layer(h, w1, stride=2, gamma=gamma1, beta=beta1)
    h = _conv_layer(h, w2, stride=2, gamma=gamma2, beta=beta2)
    h = _conv_layer(h, w3, stride=1, gamma=gamma3, beta=beta3)
    return _final_layer(h, w4, b4)
